# baseline (speedup 1.0000x reference)
import jax
import jax.numpy as jnp
from jax import lax
from jax.experimental import pallas as pl
from jax.experimental.pallas import tpu as pltpu
from jax.experimental.pallas import tpu_sc as plsc

N_COND = 26
N_CAT = 38461
EMBED = 32
B = 16384

N_TOT = B * N_COND
NW = 32
BPW = B // NW
CHUNK = N_TOT // NW
SUB = 128
NSUB = CHUNK // SUB
NBUF = 4
NROUND = NSUB // NBUF
PITCH = 133

TROWS = 999987
NSUPER = 250000
FCH = 7808
FPW = FCH // NW


def _fmt_body(catT, tail16, t128,
              tb0, tb1, tb2, tb3, sk0, sk1, ob0, ob1, i0, i1, i2, i3, o0, o1):
    tbs, sks, obs = [tb0, tb1, tb2, tb3], [sk0, sk1], [ob0, ob1]
    isems, osems = [i0, i1, i2, i3], [o0, o1]

    wid = lax.axis_index("s") * 2 + lax.axis_index("c")
    start = wid * FPW
    iota = lax.iota(jnp.int32, 16)

    def skew_base(g):
        r16 = iota + g * 16
        return lax.shift_right_logical(r16, 2) * 140 + \
            lax.bitwise_and(r16, 3) * 35

    def fire_in(k, s):
        ck = start + k
        pltpu.async_copy(catT.at[:, pl.ds(ck * 128, 128)], tbs[s], isems[s])

    def wait_in(k, s):
        ck = start + k
        pltpu.make_async_copy(catT.at[:, pl.ds(ck * 128, 128)], tbs[s],
                              isems[s]).wait()

    def fire_out(k, s):
        ck = start + k
        pltpu.async_copy(obs[s], t128.at[pl.ds(ck * 32, 32)], osems[s])

    def wait_out(k, s):
        ck = start + k
        pltpu.make_async_copy(obs[s], t128.at[pl.ds(ck * 32, 32)],
                              osems[s]).wait()

    def pass1(s4, s2):
        tb, sk = tbs[s4], sks[s2]

        def p1(e, carry):
            for g in range(8):
                plsc.store_scatter(sk, [skew_base(g) + e],
                                   tb[e, pl.ds(g * 16, 16)])
            return carry

        lax.fori_loop(0, 32, p1, 0, unroll=False)

    def pass2(s):
        sk, ob = sks[s], obs[s]

        def p2(i, carry):
            for q in range(4):
                for h in range(2):
                    ob[i, pl.ds(q * 32 + h * 16, 16)] = \
                        sk[pl.ds(i * 140 + q * 35 + h * 16, 16)]
            return carry

        lax.fori_loop(0, 32, p2, 0, unroll=False)

    def stepf(k, s4, s2, out_wait, in_fire):
        wait_in(k, s4)
        pass1(s4, s2)
        if in_fire:
            fire_in(k + 4, s4)
        if out_wait:
            wait_out(k - 2, s2)
        pass2(s2)
        fire_out(k, s2)

    for s in range(4):
        fire_in(s, s)
    stepf(0, 0, 0, False, True)
    stepf(1, 1, 1, False, True)
    stepf(2, 2, 0, True, True)
    stepf(3, 3, 1, True, True)

    def roundf(r, carry):
        for s in range(4):
            stepf(r * 4 + s, s, s % 2, True, True)
        return carry

    lax.fori_loop(1, FPW // 4 - 1, roundf, 0, unroll=False)
    k0 = FPW - 4
    for s in range(4):
        stepf(k0 + s, s, s % 2, True, False)
    wait_out(FPW - 2, 0)
    wait_out(FPW - 1, 1)

    @pl.when(wid < 4)
    def _():
        ck = FCH + wid
        pltpu.sync_copy(catT.at[:, pl.ds(ck * 128, 128)], tbs[0])
        pass1(0, 0)
        pass2(0)
        pltpu.sync_copy(obs[0], t128.at[pl.ds(ck * 32, 32)])

    @pl.when(wid == 4)
    def _():
        pltpu.sync_copy(tail16, tbs[1].at[pl.ds(0, 16)])
        pltpu.sync_copy(tbs[1].at[pl.ds(0, 16)],
                        t128.at[pl.ds(NSUPER - 16, 16)])


def _body(cond_flat, cat_t128, cond_ids, cat_ids, cat_start, out,
          cs_v, cf_v, ci_v, ca_v, idx2, st0, st1, rb0, rb1, rb2, rb3,
          g0, g1, g2, g3, o0, o1):
    rbs = [rb0, rb1, rb2, rb3]
    sts = [st0, st1]
    gsems = [g0, g1, g2, g3]
    osems = [o0, o1]

    wid = lax.axis_index("s") * 2 + lax.axis_index("c")
    base = wid * CHUNK
    wb = wid * BPW

    pltpu.sync_copy(cat_start, cs_v)
    pltpu.sync_copy(cond_flat, cf_v)
    pltpu.sync_copy(cond_ids.at[pl.ds(base, CHUNK)], ci_v)
    pltpu.sync_copy(cat_ids.at[pl.ds(base, CHUNK)], ca_v)

    iota = lax.iota(jnp.int32, 16)

    def idx_c(c, carry):
        def idx_g(g, fp16):
            cid = plsc.load_gather(ci_v, [fp16])
            cat = plsc.load_gather(ca_v, [fp16])
            s16 = plsc.load_gather(cs_v, [cid])
            full = s16 + cat
            u = c * 4 + lax.div(g, 8)
            k = lax.rem(g, 8)
            idx2[u, pl.ds(k * 16, 16)] = lax.shift_right_logical(full, 2)
            return fp16 + 16 * N_COND

        lax.fori_loop(0, BPW // 16, idx_g, iota * N_COND + c, unroll=False)
        return carry

    lax.fori_loop(0, N_COND, idx_c, 0, unroll=False)

    def fire_gather(u, s):
        pltpu.async_copy(cat_t128.at[idx2.at[u]], rbs[s], gsems[s])

    def wait_gather(u, s):
        pltpu.make_async_copy(cat_t128.at[idx2.at[u]], rbs[s],
                              gsems[s]).wait()

    def out_dst(u):
        c = lax.div(u, 4)
        blk = lax.rem(u, 4)
        return out.at[pl.ds(c, 1), :, pl.ds(wb // 128 + blk, 1), :, :]

    def fire_out(u, stg):
        pltpu.async_copy(sts[stg].at[:, :, :, :, pl.ds(0, 128)],
                         out_dst(u), osems[stg])

    def wait_out(u, stg):
        pltpu.make_async_copy(sts[stg].at[:, :, :, :, pl.ds(0, 128)],
                              out_dst(u), osems[stg]).wait()

    zero16 = jnp.zeros((16,), jnp.int32)
    e1_lo = lax.shift_right_logical(iota, 3)
    e1_hi = e1_lo + 2
    e2_16 = lax.bitwise_and(iota, 7)

    def process(u, s, stg):
        rb = rbs[s]
        st = sts[stg]
        blk = lax.rem(u, 4)
        c = lax.div(u, 4)
        fp0 = (blk * 128) * N_COND + c

        def rows(r0, carry):
            for v in range(2):
                r = r0 * 2 + v
                fp16 = jnp.full((16,), fp0 + r * N_COND, jnp.int32)
                cid16 = plsc.load_gather(ci_v, [fp16])
                cat16 = plsc.load_gather(ca_v, [fp16])
                s16 = plsc.load_gather(cs_v, [cid16])
                q16 = lax.bitwise_and(s16 + cat16, 3)
                cb16 = cid16 * EMBED
                c0 = plsc.load_gather(cf_v, [cb16 + iota])
                c1 = plsc.load_gather(cf_v, [cb16 + (iota + 16)])
                r16 = jnp.full((16,), r, jnp.int32)
                col16 = q16 * EMBED + iota
                cat0 = plsc.load_gather(rb, [r16, col16])
                cat1 = plsc.load_gather(rb, [r16, col16 + 16])
                plsc.store_scatter(st, [zero16, e1_lo, zero16, e2_16, r16],
                                   cat0 + c0)
                plsc.store_scatter(st, [zero16, e1_hi, zero16, e2_16, r16],
                                   cat1 + c1)
            return carry

        lax.fori_loop(0, SUB // 2, rows, 0, unroll=False)

    def step(u, s, stg, st_wait, prefetch):
        wait_gather(u, s)
        if st_wait:
            wait_out(u - 2, stg)
        process(u, s, stg)
        fire_out(u, stg)
        if prefetch:
            fire_gather(u + NBUF, s)

    for s in range(NBUF):
        fire_gather(s, s)
    step(0, 0, 0, False, True)
    step(1, 1, 1, False, True)
    step(2, 2, 0, True, True)
    step(3, 3, 1, True, True)

    def round_body(rnd, carry):
        u0 = rnd * NBUF
        for s in range(NBUF):
            step(u0 + s, s, s % 2, True, True)
        return carry

    lax.fori_loop(1, NROUND - 1, round_body, 0, unroll=False)

    u0 = (NROUND - 1) * NBUF
    for s in range(NBUF):
        step(u0 + s, s, s % 2, True, False)
    wait_out(NSUB - 2, 0)
    wait_out(NSUB - 1, 1)


def kernel(cond_table, cat_table, cond_ids, cat_ids, cat_start):
    cs = jnp.pad(cat_start, (0, 32 - cat_start.shape[0]))
    cond_flat = cond_table.reshape(-1)
    ci = cond_ids.reshape(-1)
    ca = cat_ids.reshape(-1)
    tail16 = jnp.pad(cat_table[FCH * 128 + 512:],
                     ((0, 13), (0, 0))).reshape(16, 128)

    mesh = plsc.VectorSubcoreMesh(core_axis_name="c", subcore_axis_name="s")

    fmt = pl.kernel(
        _fmt_body,
        out_type=jax.ShapeDtypeStruct((NSUPER, 128), jnp.float32),
        mesh=mesh,
        compiler_params=pltpu.CompilerParams(needs_layout_passes=False,
                                             use_tc_tiling_on_sc=True),
        scratch_types=(
            [pltpu.VMEM((32, 128), jnp.float32) for _ in range(4)]
            + [pltpu.VMEM((4480,), jnp.float32) for _ in range(2)]
            + [pltpu.VMEM((32, 128), jnp.float32) for _ in range(2)]
            + [pltpu.SemaphoreType.DMA for _ in range(6)]
        ),
    )

    f = pl.kernel(
        _body,
        out_type=jax.ShapeDtypeStruct((N_COND, 4, B // 128, 8, 128),
                                      jnp.float32),
        mesh=mesh,
        compiler_params=pltpu.CompilerParams(needs_layout_passes=False,
                                             use_tc_tiling_on_sc=False),
        scratch_types=(
            [
                pltpu.VMEM((32,), jnp.int32),
                pltpu.VMEM((EMBED * (N_COND + 1),), jnp.float32),
                pltpu.VMEM((CHUNK,), jnp.int32),
                pltpu.VMEM((CHUNK,), jnp.int32),
                pltpu.VMEM((NSUB, SUB), jnp.int32),
            ]
            + [pltpu.VMEM((1, 4, 1, 8, PITCH), jnp.float32)
               for _ in range(2)]
            + [pltpu.VMEM((SUB, 128), jnp.float32) for _ in range(NBUF)]
            + [pltpu.SemaphoreType.DMA for _ in range(NBUF + 2)]
        ),
    )
    t128 = fmt(cat_table.T, tail16)
    out5 = f(cond_flat, t128, ci, ca, cs)
    return out5.transpose(2, 4, 0, 1, 3).reshape(B, N_COND, EMBED)


# speedup vs baseline: 1.0758x; 1.0758x over previous
import jax
import jax.numpy as jnp
from jax import lax
from jax.experimental import pallas as pl
from jax.experimental.pallas import tpu as pltpu
from jax.experimental.pallas import tpu_sc as plsc

N_COND = 26
N_CAT = 38461
EMBED = 32
B = 16384

N_TOT = B * N_COND
NW = 32
BPW = B // NW
CHUNK = N_TOT // NW
SUB = 128
NSUB = CHUNK // SUB
NBUF = 4
NROUND = NSUB // NBUF
PITCH = 133

TROWS = 999987
NSUPER = 250000
FCH = 7808
FPW = FCH // NW


def _fmt_body(catT, tail16, t128,
              tb0, tb1, tb2, tb3, sk0, sk1, ob0, ob1, i0, i1, i2, i3, o0, o1):
    tbs, sks, obs = [tb0, tb1, tb2, tb3], [sk0, sk1], [ob0, ob1]
    isems, osems = [i0, i1, i2, i3], [o0, o1]

    wid = lax.axis_index("s") * 2 + lax.axis_index("c")
    start = wid * FPW
    iota = lax.iota(jnp.int32, 16)

    def skew_base(g):
        r16 = iota + g * 16
        return lax.shift_right_logical(r16, 2) * 140 + \
            lax.bitwise_and(r16, 3) * 35

    def fire_in(k, s):
        ck = start + k
        pltpu.async_copy(catT.at[:, pl.ds(ck * 128, 128)], tbs[s], isems[s])

    def wait_in(k, s):
        ck = start + k
        pltpu.make_async_copy(catT.at[:, pl.ds(ck * 128, 128)], tbs[s],
                              isems[s]).wait()

    def fire_out(k, s):
        ck = start + k
        pltpu.async_copy(obs[s], t128.at[pl.ds(ck * 32, 32)], osems[s])

    def wait_out(k, s):
        ck = start + k
        pltpu.make_async_copy(obs[s], t128.at[pl.ds(ck * 32, 32)],
                              osems[s]).wait()

    def pass1(s4, s2):
        tb, sk = tbs[s4], sks[s2]

        def p1(e, carry):
            for g in range(8):
                plsc.store_scatter(sk, [skew_base(g) + e],
                                   tb[e, pl.ds(g * 16, 16)])
            return carry

        lax.fori_loop(0, 32, p1, 0, unroll=False)

    def pass2(s):
        sk, ob = sks[s], obs[s]

        def p2(i, carry):
            for q in range(4):
                for h in range(2):
                    ob[i, pl.ds(q * 32 + h * 16, 16)] = \
                        sk[pl.ds(i * 140 + q * 35 + h * 16, 16)]
            return carry

        lax.fori_loop(0, 32, p2, 0, unroll=False)

    def stepf(k, s4, s2, out_wait, in_fire):
        wait_in(k, s4)
        pass1(s4, s2)
        if in_fire:
            fire_in(k + 4, s4)
        if out_wait:
            wait_out(k - 2, s2)
        pass2(s2)
        fire_out(k, s2)

    for s in range(4):
        fire_in(s, s)
    stepf(0, 0, 0, False, True)
    stepf(1, 1, 1, False, True)
    stepf(2, 2, 0, True, True)
    stepf(3, 3, 1, True, True)

    def roundf(r, carry):
        for s in range(4):
            stepf(r * 4 + s, s, s % 2, True, True)
        return carry

    lax.fori_loop(1, FPW // 4 - 1, roundf, 0, unroll=False)
    k0 = FPW - 4
    for s in range(4):
        stepf(k0 + s, s, s % 2, True, False)
    wait_out(FPW - 2, 0)
    wait_out(FPW - 1, 1)

    @pl.when(wid < 4)
    def _():
        ck = FCH + wid
        pltpu.sync_copy(catT.at[:, pl.ds(ck * 128, 128)], tbs[0])
        pass1(0, 0)
        pass2(0)
        pltpu.sync_copy(obs[0], t128.at[pl.ds(ck * 32, 32)])

    @pl.when(wid == 4)
    def _():
        pltpu.sync_copy(tail16, tbs[1].at[pl.ds(0, 16)])
        pltpu.sync_copy(tbs[1].at[pl.ds(0, 16)],
                        t128.at[pl.ds(NSUPER - 16, 16)])


def _body(cond_flat, cat_t128, cond_ids, cat_ids, cat_start, out,
          cs_v, cf_v, ci_v, ca_v, idx2, st0, st1, rb0, rb1, rb2, rb3,
          g0, g1, g2, g3, o0, o1):
    rbs = [rb0, rb1, rb2, rb3]
    sts = [st0, st1]
    gsems = [g0, g1, g2, g3]
    osems = [o0, o1]

    wid = lax.axis_index("s") * 2 + lax.axis_index("c")
    base = wid * CHUNK
    wb = wid * BPW

    pltpu.sync_copy(cat_start, cs_v)
    pltpu.sync_copy(cond_flat, cf_v)
    pltpu.sync_copy(cond_ids.at[pl.ds(base, CHUNK)], ci_v)
    pltpu.sync_copy(cat_ids.at[pl.ds(base, CHUNK)], ca_v)

    iota = lax.iota(jnp.int32, 16)

    def idx_c(c, carry):
        def idx_g(g, fp16):
            cid = plsc.load_gather(ci_v, [fp16])
            cat = plsc.load_gather(ca_v, [fp16])
            s16 = plsc.load_gather(cs_v, [cid])
            full = s16 + cat
            u = c * 4 + lax.div(g, 8)
            k = lax.rem(g, 8)
            idx2[u, pl.ds(k * 16, 16)] = lax.shift_right_logical(full, 2)
            return fp16 + 16 * N_COND

        lax.fori_loop(0, BPW // 16, idx_g, iota * N_COND + c, unroll=False)
        return carry

    lax.fori_loop(0, N_COND, idx_c, 0, unroll=False)

    def fire_gather(u, s):
        pltpu.async_copy(cat_t128.at[idx2.at[u]], rbs[s], gsems[s])

    def wait_gather(u, s):
        pltpu.make_async_copy(cat_t128.at[idx2.at[u]], rbs[s],
                              gsems[s]).wait()

    def out_dst(u):
        c = lax.div(u, 4)
        blk = lax.rem(u, 4)
        return out.at[pl.ds(c, 1), :, pl.ds(wb // 128 + blk, 1), :, :]

    def fire_out(u, stg):
        pltpu.async_copy(sts[stg].at[:, :, :, :, pl.ds(0, 128)],
                         out_dst(u), osems[stg])

    def wait_out(u, stg):
        pltpu.make_async_copy(sts[stg].at[:, :, :, :, pl.ds(0, 128)],
                              out_dst(u), osems[stg]).wait()

    zero16 = jnp.zeros((16,), jnp.int32)
    e1_lo = lax.shift_right_logical(iota, 3)
    e1_hi = e1_lo + 2
    e2_16 = lax.bitwise_and(iota, 7)

    def process(u, s, stg):
        rb = rbs[s]
        st = sts[stg]
        blk = lax.rem(u, 4)
        c = lax.div(u, 4)
        fp0 = (blk * 128) * N_COND + c

        def rows(r0, carry):
            for v in range(2):
                r = r0 * 2 + v
                fp16 = jnp.full((16,), fp0 + r * N_COND, jnp.int32)
                cid16 = plsc.load_gather(ci_v, [fp16])
                cat16 = plsc.load_gather(ca_v, [fp16])
                q16 = lax.bitwise_and(cid16 + cat16, 3)
                cb16 = cid16 * EMBED
                c0 = plsc.load_gather(cf_v, [cb16 + iota])
                c1 = plsc.load_gather(cf_v, [cb16 + (iota + 16)])
                r16 = jnp.full((16,), r, jnp.int32)
                col16 = q16 * EMBED + iota
                cat0 = plsc.load_gather(rb, [r16, col16])
                cat1 = plsc.load_gather(rb, [r16, col16 + 16])
                plsc.store_scatter(st, [zero16, e1_lo, zero16, e2_16, r16],
                                   cat0 + c0)
                plsc.store_scatter(st, [zero16, e1_hi, zero16, e2_16, r16],
                                   cat1 + c1)
            return carry

        lax.fori_loop(0, SUB // 2, rows, 0, unroll=False)

    def step(u, s, stg, st_wait, prefetch):
        wait_gather(u, s)
        if st_wait:
            wait_out(u - 2, stg)
        process(u, s, stg)
        fire_out(u, stg)
        if prefetch:
            fire_gather(u + NBUF, s)

    for s in range(NBUF):
        fire_gather(s, s)
    step(0, 0, 0, False, True)
    step(1, 1, 1, False, True)
    step(2, 2, 0, True, True)
    step(3, 3, 1, True, True)

    def round_body(rnd, carry):
        u0 = rnd * NBUF
        for s in range(NBUF):
            step(u0 + s, s, s % 2, True, True)
        return carry

    lax.fori_loop(1, NROUND - 1, round_body, 0, unroll=False)

    u0 = (NROUND - 1) * NBUF
    for s in range(NBUF):
        step(u0 + s, s, s % 2, True, False)
    wait_out(NSUB - 2, 0)
    wait_out(NSUB - 1, 1)


def kernel(cond_table, cat_table, cond_ids, cat_ids, cat_start):
    cs = jnp.pad(cat_start, (0, 32 - cat_start.shape[0]))
    cond_flat = cond_table.reshape(-1)
    ci = cond_ids.reshape(-1)
    ca = cat_ids.reshape(-1)
    tail16 = jnp.pad(cat_table[FCH * 128 + 512:],
                     ((0, 13), (0, 0))).reshape(16, 128)

    mesh = plsc.VectorSubcoreMesh(core_axis_name="c", subcore_axis_name="s")

    fmt = pl.kernel(
        _fmt_body,
        out_type=jax.ShapeDtypeStruct((NSUPER, 128), jnp.float32),
        mesh=mesh,
        compiler_params=pltpu.CompilerParams(needs_layout_passes=False,
                                             use_tc_tiling_on_sc=True),
        scratch_types=(
            [pltpu.VMEM((32, 128), jnp.float32) for _ in range(4)]
            + [pltpu.VMEM((4480,), jnp.float32) for _ in range(2)]
            + [pltpu.VMEM((32, 128), jnp.float32) for _ in range(2)]
            + [pltpu.SemaphoreType.DMA for _ in range(6)]
        ),
    )

    f = pl.kernel(
        _body,
        out_type=jax.ShapeDtypeStruct((N_COND, 4, B // 128, 8, 128),
                                      jnp.float32),
        mesh=mesh,
        compiler_params=pltpu.CompilerParams(needs_layout_passes=False,
                                             use_tc_tiling_on_sc=False),
        scratch_types=(
            [
                pltpu.VMEM((32,), jnp.int32),
                pltpu.VMEM((EMBED * (N_COND + 1),), jnp.float32),
                pltpu.VMEM((CHUNK,), jnp.int32),
                pltpu.VMEM((CHUNK,), jnp.int32),
                pltpu.VMEM((NSUB, SUB), jnp.int32),
            ]
            + [pltpu.VMEM((1, 4, 1, 8, PITCH), jnp.float32)
               for _ in range(2)]
            + [pltpu.VMEM((SUB, 128), jnp.float32) for _ in range(NBUF)]
            + [pltpu.SemaphoreType.DMA for _ in range(NBUF + 2)]
        ),
    )
    t128 = fmt(cat_table.T, tail16)
    out5 = f(cond_flat, t128, ci, ca, cs)
    return out5.transpose(2, 4, 0, 1, 3).reshape(B, N_COND, EMBED)


# speedup vs baseline: 1.1067x; 1.0286x over previous
import jax
import jax.numpy as jnp
from jax import lax
from jax.experimental import pallas as pl
from jax.experimental.pallas import tpu as pltpu
from jax.experimental.pallas import tpu_sc as plsc

N_COND = 26
N_CAT = 38461
EMBED = 32
B = 16384

N_TOT = B * N_COND
NW = 32
BPW = B // NW
CHUNK = N_TOT // NW
SUB = 128
NSUB = CHUNK // SUB
NBUF = 4
NROUND = NSUB // NBUF
PITCH = 133

TROWS = 999987
NSUPER = 250000
FCH = 7808
FPW = FCH // NW


def _fmt_body(catT, tail16, t128,
              tb0, tb1, tb2, tb3, sk0, sk1, ob0, ob1, i0, i1, i2, i3, o0, o1):
    tbs, sks, obs = [tb0, tb1, tb2, tb3], [sk0, sk1], [ob0, ob1]
    isems, osems = [i0, i1, i2, i3], [o0, o1]

    wid = lax.axis_index("s") * 2 + lax.axis_index("c")
    start = wid * FPW
    iota = lax.iota(jnp.int32, 16)

    def skew_base(g):
        r16 = iota + g * 16
        return lax.shift_right_logical(r16, 2) * 140 + \
            lax.bitwise_and(r16, 3) * 35

    def fire_in(k, s):
        ck = start + k
        pltpu.async_copy(catT.at[:, pl.ds(ck * 128, 128)], tbs[s], isems[s])

    def wait_in(k, s):
        ck = start + k
        pltpu.make_async_copy(catT.at[:, pl.ds(ck * 128, 128)], tbs[s],
                              isems[s]).wait()

    def fire_out(k, s):
        ck = start + k
        pltpu.async_copy(obs[s], t128.at[pl.ds(ck * 32, 32)], osems[s])

    def wait_out(k, s):
        ck = start + k
        pltpu.make_async_copy(obs[s], t128.at[pl.ds(ck * 32, 32)],
                              osems[s]).wait()

    def pass1(s4, s2):
        tb, sk = tbs[s4], sks[s2]

        def p1(e, carry):
            for g in range(8):
                plsc.store_scatter(sk, [skew_base(g) + e],
                                   tb[e, pl.ds(g * 16, 16)])
            return carry

        lax.fori_loop(0, 32, p1, 0, unroll=False)

    def pass2(s):
        sk, ob = sks[s], obs[s]

        def p2(i, carry):
            for q in range(4):
                for h in range(2):
                    ob[i, pl.ds(q * 32 + h * 16, 16)] = \
                        sk[pl.ds(i * 140 + q * 35 + h * 16, 16)]
            return carry

        lax.fori_loop(0, 32, p2, 0, unroll=False)

    def stepf(k, s4, s2, out_wait, in_fire):
        wait_in(k, s4)
        pass1(s4, s2)
        if in_fire:
            fire_in(k + 4, s4)
        if out_wait:
            wait_out(k - 2, s2)
        pass2(s2)
        fire_out(k, s2)

    for s in range(4):
        fire_in(s, s)
    stepf(0, 0, 0, False, True)
    stepf(1, 1, 1, False, True)
    stepf(2, 2, 0, True, True)
    stepf(3, 3, 1, True, True)

    def roundf(r, carry):
        for s in range(4):
            stepf(r * 4 + s, s, s % 2, True, True)
        return carry

    lax.fori_loop(1, FPW // 4 - 1, roundf, 0, unroll=False)
    k0 = FPW - 4
    for s in range(4):
        stepf(k0 + s, s, s % 2, True, False)
    wait_out(FPW - 2, 0)
    wait_out(FPW - 1, 1)

    @pl.when(wid < 4)
    def _():
        ck = FCH + wid
        pltpu.sync_copy(catT.at[:, pl.ds(ck * 128, 128)], tbs[0])
        pass1(0, 0)
        pass2(0)
        pltpu.sync_copy(obs[0], t128.at[pl.ds(ck * 32, 32)])

    @pl.when(wid == 4)
    def _():
        pltpu.sync_copy(tail16, tbs[1].at[pl.ds(0, 16)])
        pltpu.sync_copy(tbs[1].at[pl.ds(0, 16)],
                        t128.at[pl.ds(NSUPER - 16, 16)])


def _body(cond_flat, cat_t128, cond_ids, cat_ids, cat_start, out,
          cs_v, cf_v, ci_v, ca_v, idx2, pv, st0, st1, rb0, rb1, rb2, rb3,
          g0, g1, g2, g3, o0, o1):
    rbs = [rb0, rb1, rb2, rb3]
    sts = [st0, st1]
    gsems = [g0, g1, g2, g3]
    osems = [o0, o1]

    wid = lax.axis_index("s") * 2 + lax.axis_index("c")
    base = wid * CHUNK
    wb = wid * BPW

    pltpu.sync_copy(cat_start, cs_v)
    pltpu.sync_copy(cond_flat, cf_v)
    pltpu.sync_copy(cond_ids.at[pl.ds(base, CHUNK)], ci_v)
    pltpu.sync_copy(cat_ids.at[pl.ds(base, CHUNK)], ca_v)

    iota = lax.iota(jnp.int32, 16)

    def idx_c(c, carry):
        def idx_g(g, fp16):
            cid = plsc.load_gather(ci_v, [fp16])
            cat = plsc.load_gather(ca_v, [fp16])
            s16 = plsc.load_gather(cs_v, [cid])
            full = s16 + cat
            u = c * 4 + lax.div(g, 8)
            k = lax.rem(g, 8)
            idx2[u, pl.ds(k * 16, 16)] = lax.shift_right_logical(full, 2)
            pv[u, pl.ds(k * 16, 16)] = cid * 128 + lax.bitwise_and(full, 3)
            return fp16 + 16 * N_COND

        lax.fori_loop(0, BPW // 16, idx_g, iota * N_COND + c, unroll=False)
        return carry

    lax.fori_loop(0, N_COND, idx_c, 0, unroll=False)

    def fire_gather(u, s):
        pltpu.async_copy(cat_t128.at[idx2.at[u]], rbs[s], gsems[s])

    def wait_gather(u, s):
        pltpu.make_async_copy(cat_t128.at[idx2.at[u]], rbs[s],
                              gsems[s]).wait()

    def out_dst(u):
        c = lax.div(u, 4)
        blk = lax.rem(u, 4)
        return out.at[pl.ds(c, 1), :, pl.ds(wb // 128 + blk, 1), :, :]

    def fire_out(u, stg):
        pltpu.async_copy(sts[stg].at[:, :, :, :, pl.ds(0, 128)],
                         out_dst(u), osems[stg])

    def wait_out(u, stg):
        pltpu.make_async_copy(sts[stg].at[:, :, :, :, pl.ds(0, 128)],
                              out_dst(u), osems[stg]).wait()

    zero16 = jnp.zeros((16,), jnp.int32)
    e1_lo = lax.shift_right_logical(iota, 3)
    e1_hi = e1_lo + 2
    e2_16 = lax.bitwise_and(iota, 7)

    def process(u, s, stg):
        rb = rbs[s]
        st = sts[stg]
        u16 = jnp.full((16,), u, jnp.int32)

        def rows(r0, carry):
            for v in range(2):
                r = r0 * 2 + v
                v16 = plsc.load_gather(pv, [u16,
                                            jnp.full((16,), r, jnp.int32)])
                cb16 = lax.shift_right_logical(v16, 2)
                q16 = lax.bitwise_and(v16, 3)
                c0 = plsc.load_gather(cf_v, [cb16 + iota])
                c1 = plsc.load_gather(cf_v, [cb16 + (iota + 16)])
                r16 = jnp.full((16,), r, jnp.int32)
                col16 = q16 * EMBED + iota
                cat0 = plsc.load_gather(rb, [r16, col16])
                cat1 = plsc.load_gather(rb, [r16, col16 + 16])
                plsc.store_scatter(st, [zero16, e1_lo, zero16, e2_16, r16],
                                   cat0 + c0)
                plsc.store_scatter(st, [zero16, e1_hi, zero16, e2_16, r16],
                                   cat1 + c1)
            return carry

        lax.fori_loop(0, SUB // 2, rows, 0, unroll=False)

    def step(u, s, stg, st_wait, prefetch):
        wait_gather(u, s)
        if st_wait:
            wait_out(u - 2, stg)
        process(u, s, stg)
        fire_out(u, stg)
        if prefetch:
            fire_gather(u + NBUF, s)

    for s in range(NBUF):
        fire_gather(s, s)
    step(0, 0, 0, False, True)
    step(1, 1, 1, False, True)
    step(2, 2, 0, True, True)
    step(3, 3, 1, True, True)

    def round_body(rnd, carry):
        u0 = rnd * NBUF
        for s in range(NBUF):
            step(u0 + s, s, s % 2, True, True)
        return carry

    lax.fori_loop(1, NROUND - 1, round_body, 0, unroll=False)

    u0 = (NROUND - 1) * NBUF
    for s in range(NBUF):
        step(u0 + s, s, s % 2, True, False)
    wait_out(NSUB - 2, 0)
    wait_out(NSUB - 1, 1)


def kernel(cond_table, cat_table, cond_ids, cat_ids, cat_start):
    cs = jnp.pad(cat_start, (0, 32 - cat_start.shape[0]))
    cond_flat = cond_table.reshape(-1)
    ci = cond_ids.reshape(-1)
    ca = cat_ids.reshape(-1)
    tail16 = jnp.pad(cat_table[FCH * 128 + 512:],
                     ((0, 13), (0, 0))).reshape(16, 128)

    mesh = plsc.VectorSubcoreMesh(core_axis_name="c", subcore_axis_name="s")

    fmt = pl.kernel(
        _fmt_body,
        out_type=jax.ShapeDtypeStruct((NSUPER, 128), jnp.float32),
        mesh=mesh,
        compiler_params=pltpu.CompilerParams(needs_layout_passes=False,
                                             use_tc_tiling_on_sc=True),
        scratch_types=(
            [pltpu.VMEM((32, 128), jnp.float32) for _ in range(4)]
            + [pltpu.VMEM((4480,), jnp.float32) for _ in range(2)]
            + [pltpu.VMEM((32, 128), jnp.float32) for _ in range(2)]
            + [pltpu.SemaphoreType.DMA for _ in range(6)]
        ),
    )

    f = pl.kernel(
        _body,
        out_type=jax.ShapeDtypeStruct((N_COND, 4, B // 128, 8, 128),
                                      jnp.float32),
        mesh=mesh,
        compiler_params=pltpu.CompilerParams(needs_layout_passes=False,
                                             use_tc_tiling_on_sc=False),
        scratch_types=(
            [
                pltpu.VMEM((32,), jnp.int32),
                pltpu.VMEM((EMBED * (N_COND + 1),), jnp.float32),
                pltpu.VMEM((CHUNK,), jnp.int32),
                pltpu.VMEM((CHUNK,), jnp.int32),
                pltpu.VMEM((NSUB, SUB), jnp.int32),
                pltpu.VMEM((NSUB, SUB), jnp.int32),
            ]
            + [pltpu.VMEM((1, 4, 1, 8, PITCH), jnp.float32)
               for _ in range(2)]
            + [pltpu.VMEM((SUB, 128), jnp.float32) for _ in range(NBUF)]
            + [pltpu.SemaphoreType.DMA for _ in range(NBUF + 2)]
        ),
    )
    t128 = fmt(cat_table.T, tail16)
    out5 = f(cond_flat, t128, ci, ca, cs)
    return out5.transpose(2, 4, 0, 1, 3).reshape(B, N_COND, EMBED)
